# baseline (speedup 1.0000x reference)
import functools

import jax
import jax.numpy as jnp
from jax import lax
from jax.experimental import pallas as pl
from jax.experimental.pallas import tpu as pltpu
from jax.experimental.pallas import tpu_sc as plsc

_SC_INFO = plsc.get_sparse_core_info()
_NC = _SC_INFO.num_cores


def _x_kernel(res_ref, x_ref, xi_ref):
    xi_ref[...] = x_ref[...] + res_ref[0]


def kernel(x, ptx, bs, height, width, point_edges, point_src_dirs, point_tgt_dirs):
    C, H, W = x.shape[1], x.shape[2], x.shape[3]
    n_pts = ptx.shape[0]
    residual = (
        (jnp.asarray(bs) - 1) + (jnp.asarray(height) - H) + (jnp.asarray(width) - W)
    ).astype(x.dtype)
    res = residual.reshape(1)

    n_workers = 4
    rows = n_pts // n_workers
    mesh = plsc.VectorSubcoreMesh(core_axis_name="c", subcore_axis_name="s")

    @functools.partial(
        pl.kernel,
        mesh=mesh,
        out_type=jax.ShapeDtypeStruct((n_pts, C), ptx.dtype),
    )
    def _sc_copy(ptx_hbm, out_hbm):
        wid = lax.axis_index("s") * _NC + lax.axis_index("c")

        @pl.when(wid < n_workers)
        def _():
            base = wid * rows
            pltpu.sync_copy(
                ptx_hbm.at[pl.ds(base, rows)], out_hbm.at[pl.ds(base, rows)]
            )

    G = 4
    xb = C // G
    xi = pl.pallas_call(
        _x_kernel,
        grid=(G,),
        in_specs=[
            pl.BlockSpec(memory_space=pltpu.SMEM),
            pl.BlockSpec((1, xb, H, W), lambda i: (0, i, 0, 0)),
        ],
        out_specs=pl.BlockSpec((1, xb, H, W), lambda i: (0, i, 0, 0)),
        out_shape=jax.ShapeDtypeStruct((1, C, H, W), x.dtype),
    )(res, x)

    return (xi, _sc_copy(ptx))


# speedup vs baseline: 16.3697x; 16.3697x over previous
import jax
import jax.numpy as jnp
from jax.experimental import pallas as pl
from jax.experimental.pallas import tpu as pltpu


def _stream_kernel(res_ref, x_ref, ptx_ref, xi_ref, ptx_out_ref):
    xi_ref[...] = x_ref[...] + res_ref[0]
    ptx_out_ref[...] = ptx_ref[...]


def kernel(x, ptx, bs, height, width, point_edges, point_src_dirs, point_tgt_dirs):
    C, H, W = x.shape[1], x.shape[2], x.shape[3]
    n_pts = ptx.shape[0]
    residual = (
        (jnp.asarray(bs) - 1) + (jnp.asarray(height) - H) + (jnp.asarray(width) - W)
    ).astype(x.dtype)
    res = residual.reshape(1)

    G = 4
    xb = C // G
    pb = n_pts // 4

    xi, ptx_out = pl.pallas_call(
        _stream_kernel,
        grid=(G,),
        in_specs=[
            pl.BlockSpec(memory_space=pltpu.SMEM),
            pl.BlockSpec((1, xb, H, W), lambda i: (0, i, 0, 0)),
            pl.BlockSpec((pb, C), lambda i: (i, 0)),
        ],
        out_specs=[
            pl.BlockSpec((1, xb, H, W), lambda i: (0, i, 0, 0)),
            pl.BlockSpec((pb, C), lambda i: (i, 0)),
        ],
        out_shape=[
            jax.ShapeDtypeStruct((1, C, H, W), x.dtype),
            jax.ShapeDtypeStruct((n_pts, C), ptx.dtype),
        ],
    )(res, x, ptx)

    return (xi, ptx_out)
